# Initial kernel scaffold; baseline (speedup 1.0000x reference)
import jax
import jax.numpy as jnp
from jax.experimental import pallas as pl


def kernel(item_id, item_gics, item_name_tokens, id_table, gics_table, name_table):
    raise NotImplementedError("write your pallas kernel here")


# speedup vs baseline: 5.1066x; 5.1066x over previous
import functools

import jax
import jax.numpy as jnp
from jax import lax
from jax.experimental import pallas as pl
from jax.experimental.pallas import tpu as pltpu
from jax.experimental.pallas import tpu_sc as plsc

NC = 2
NS = 16
NW = NC * NS
LANES = 16

B = 16384
N_ITEMS = 5
NAME_LEN = 8
ROWS = B * N_ITEMS
ITEMS_PW = ROWS // NW
CHUNK = 256
NCH = ITEMS_PW // CHUNK
IW = 128
D_ID = 16
D_GICS = 16
D_NAME = 32
D_OUT = D_ID + D_GICS + D_NAME


def _body(id_idx, gics_idx, name_idx, id_table, gics_table, name_table,
          out, idx_id_v, idx_gics_v, idx_name_v, id_rows, gics_rows,
          tok_rows, acc, sem):
    wid = lax.axis_index("s") * NC + lax.axis_index("c")

    def chunk_body(ci, _):
        row_i = wid * (ITEMS_PW // IW) + ci * (CHUNK // IW)
        row_n = wid * (ITEMS_PW * NAME_LEN // IW) + ci * (CHUNK * NAME_LEN // IW)
        out_base = wid * ITEMS_PW + ci * CHUNK

        pltpu.sync_copy(id_idx.at[pl.ds(row_i, CHUNK // IW)], idx_id_v)
        pltpu.sync_copy(gics_idx.at[pl.ds(row_i, CHUNK // IW)], idx_gics_v)
        pltpu.sync_copy(name_idx.at[pl.ds(row_n, CHUNK * NAME_LEN // IW)],
                        idx_name_v)

        handles = []
        for j in range(CHUNK // IW):
            handles.append(pltpu.async_copy(
                id_table.at[idx_id_v.at[j]],
                id_rows.at[pl.ds(j * IW, IW)], sem))
            handles.append(pltpu.async_copy(
                gics_table.at[idx_gics_v.at[j]],
                gics_rows.at[pl.ds(j * IW, IW)], sem))
        for j in range(CHUNK * NAME_LEN // IW):
            handles.append(pltpu.async_copy(
                name_table.at[idx_name_v.at[j]],
                tok_rows.at[pl.ds(j * IW, IW)], sem))
        for h in handles:
            h.wait()

        def pool_body(i, _):
            s0 = tok_rows[i * NAME_LEN, pl.ds(0, LANES)]
            s1 = tok_rows[i * NAME_LEN, pl.ds(LANES, LANES)]
            for t in range(1, NAME_LEN):
                s0 = s0 + tok_rows[i * NAME_LEN + t, pl.ds(0, LANES)]
                s1 = s1 + tok_rows[i * NAME_LEN + t, pl.ds(LANES, LANES)]
            acc[i, pl.ds(0, LANES)] = s0 * (1.0 / NAME_LEN)
            acc[i, pl.ds(LANES, LANES)] = s1 * (1.0 / NAME_LEN)
            return 0

        lax.fori_loop(0, CHUNK, pool_body, 0)

        pltpu.sync_copy(id_rows, out.at[pl.ds(out_base, CHUNK), pl.ds(0, D_ID)])
        pltpu.sync_copy(gics_rows,
                        out.at[pl.ds(out_base, CHUNK), pl.ds(D_ID, D_GICS)])
        pltpu.sync_copy(acc,
                        out.at[pl.ds(out_base, CHUNK),
                               pl.ds(D_ID + D_GICS, D_NAME)])
        return 0

    lax.fori_loop(0, NCH, chunk_body, 0)


@jax.jit
def kernel(item_id, item_gics, item_name_tokens, id_table, gics_table,
           name_table):
    id_idx = jnp.asarray(item_id, jnp.int32).reshape(ROWS // IW, IW)
    gics_idx = jnp.asarray(item_gics, jnp.int32).reshape(ROWS // IW, IW)
    name_idx = jnp.asarray(item_name_tokens, jnp.int32).reshape(
        ROWS * NAME_LEN // IW, IW)

    kfn = pl.kernel(
        _body,
        out_type=jax.ShapeDtypeStruct((ROWS, D_OUT), jnp.float32),
        mesh=plsc.VectorSubcoreMesh(core_axis_name="c", subcore_axis_name="s"),
        compiler_params=pltpu.CompilerParams(use_tc_tiling_on_sc=False),
        scratch_types=[
            pltpu.VMEM((CHUNK // IW, IW), jnp.int32),
            pltpu.VMEM((CHUNK // IW, IW), jnp.int32),
            pltpu.VMEM((CHUNK * NAME_LEN // IW, IW), jnp.int32),
            pltpu.VMEM((CHUNK, D_ID), jnp.float32),
            pltpu.VMEM((CHUNK, D_GICS), jnp.float32),
            pltpu.VMEM((CHUNK * NAME_LEN, D_NAME), jnp.float32),
            pltpu.VMEM((CHUNK, D_NAME), jnp.float32),
            pltpu.SemaphoreType.DMA,
        ],
    )
    out = kfn(id_idx, gics_idx, name_idx, id_table, gics_table, name_table)
    return out.reshape(B, N_ITEMS, D_OUT)


# speedup vs baseline: 5.5701x; 1.0908x over previous
import jax
import jax.numpy as jnp
from jax import lax
from jax.experimental import pallas as pl
from jax.experimental.pallas import tpu as pltpu
from jax.experimental.pallas import tpu_sc as plsc

NC = 2
NS = 16
NW = NC * NS
LANES = 16

B = 16384
N_ITEMS = 5
NAME_LEN = 8
ROWS = B * N_ITEMS
ITEMS_PW = ROWS // NW
IW = 128
CHUNK = IW
NCH = ITEMS_PW // CHUNK
D_ID = 16
D_GICS = 16
D_NAME = 32
D_OUT = D_ID + D_GICS + D_NAME


def _body(id_idx, gics_idx, name_idx, id_table, gics_table, name_table,
          out, idx_id_v, idx_gics_v, idx_name_v,
          id_r0, id_r1, gc_r0, gc_r1, tok0, tok1, acc0, acc1,
          sem_g0, sem_g1, sem_o0, sem_o1):
    wid = lax.axis_index("s") * NC + lax.axis_index("c")
    bufs = ((id_r0, gc_r0, tok0, acc0, sem_g0, sem_o0),
            (id_r1, gc_r1, tok1, acc1, sem_g1, sem_o1))

    pltpu.sync_copy(id_idx.at[pl.ds(wid * NCH, NCH)], idx_id_v)
    pltpu.sync_copy(gics_idx.at[pl.ds(wid * NCH, NCH)], idx_gics_v)
    pltpu.sync_copy(name_idx.at[pl.ds(wid * NCH * NAME_LEN, NCH * NAME_LEN)],
                    idx_name_v)

    def gather_descs(ci, b):
        idr, gcr, tok, _, semg, _ = bufs[b]
        ds = [pltpu.make_async_copy(id_table.at[idx_id_v.at[ci]], idr, semg),
              pltpu.make_async_copy(gics_table.at[idx_gics_v.at[ci]], gcr,
                                    semg)]
        for j in range(NAME_LEN):
            ds.append(pltpu.make_async_copy(
                name_table.at[idx_name_v.at[ci * NAME_LEN + j]],
                tok.at[pl.ds(j * IW, IW)], semg))
        return ds

    def out_descs(ci, b):
        idr, gcr, _, acc, _, semo = bufs[b]
        base = wid * ITEMS_PW + ci * CHUNK
        return [
            pltpu.make_async_copy(
                idr, out.at[pl.ds(base, CHUNK), pl.ds(0, D_ID)], semo),
            pltpu.make_async_copy(
                gcr, out.at[pl.ds(base, CHUNK), pl.ds(D_ID, D_GICS)], semo),
            pltpu.make_async_copy(
                acc, out.at[pl.ds(base, CHUNK), pl.ds(D_ID + D_GICS, D_NAME)],
                semo),
        ]

    def pool(b):
        _, _, tok, acc, _, _ = bufs[b]

        def pool_body(i, _):
            s0 = tok[i * NAME_LEN, pl.ds(0, LANES)]
            s1 = tok[i * NAME_LEN, pl.ds(LANES, LANES)]
            for t in range(1, NAME_LEN):
                s0 = s0 + tok[i * NAME_LEN + t, pl.ds(0, LANES)]
                s1 = s1 + tok[i * NAME_LEN + t, pl.ds(LANES, LANES)]
            acc[i, pl.ds(0, LANES)] = s0 * (1.0 / NAME_LEN)
            acc[i, pl.ds(LANES, LANES)] = s1 * (1.0 / NAME_LEN)
            return 0

        lax.fori_loop(0, CHUNK, pool_body, 0, unroll=2)

    for d in gather_descs(0, 0):
        d.start()

    def super_body(s, _):
        for b in range(2):
            ci = 2 * s + b
            nb = 1 - b

            @pl.when(ci + 1 < NCH)
            def _issue_next():
                @pl.when(ci >= 1)
                def _drain_prev_out():
                    for d in out_descs(ci - 1, nb):
                        d.wait()

                for d in gather_descs(ci + 1, nb):
                    d.start()

            for d in gather_descs(ci, b):
                d.wait()
            pool(b)
            for d in out_descs(ci, b):
                d.start()
        return 0

    lax.fori_loop(0, NCH // 2, super_body, 0)
    for d in out_descs(NCH - 2, 0):
        d.wait()
    for d in out_descs(NCH - 1, 1):
        d.wait()


@jax.jit
def kernel(item_id, item_gics, item_name_tokens, id_table, gics_table,
           name_table):
    id_idx = jnp.asarray(item_id, jnp.int32).reshape(ROWS // IW, IW)
    gics_idx = jnp.asarray(item_gics, jnp.int32).reshape(ROWS // IW, IW)
    name_idx = jnp.asarray(item_name_tokens, jnp.int32).reshape(
        ROWS * NAME_LEN // IW, IW)

    kfn = pl.kernel(
        _body,
        out_type=jax.ShapeDtypeStruct((ROWS, D_OUT), jnp.float32),
        mesh=plsc.VectorSubcoreMesh(core_axis_name="c", subcore_axis_name="s"),
        compiler_params=pltpu.CompilerParams(use_tc_tiling_on_sc=False),
        scratch_types=[
            pltpu.VMEM((NCH, IW), jnp.int32),
            pltpu.VMEM((NCH, IW), jnp.int32),
            pltpu.VMEM((NCH * NAME_LEN, IW), jnp.int32),
            pltpu.VMEM((CHUNK, D_ID), jnp.float32),
            pltpu.VMEM((CHUNK, D_ID), jnp.float32),
            pltpu.VMEM((CHUNK, D_GICS), jnp.float32),
            pltpu.VMEM((CHUNK, D_GICS), jnp.float32),
            pltpu.VMEM((CHUNK * NAME_LEN, D_NAME), jnp.float32),
            pltpu.VMEM((CHUNK * NAME_LEN, D_NAME), jnp.float32),
            pltpu.VMEM((CHUNK, D_NAME), jnp.float32),
            pltpu.VMEM((CHUNK, D_NAME), jnp.float32),
            pltpu.SemaphoreType.DMA,
            pltpu.SemaphoreType.DMA,
            pltpu.SemaphoreType.DMA,
            pltpu.SemaphoreType.DMA,
        ],
    )
    out = kfn(id_idx, gics_idx, name_idx, id_table, gics_table, name_table)
    return out.reshape(B, N_ITEMS, D_OUT)
